# baseline (speedup 1.0000x reference)
import functools

import jax
import jax.numpy as jnp
from jax.experimental import pallas as pl
from jax.experimental.pallas import tpu as pltpu

MAX_LEN = 2048
EMBED_DIM = 768
BLOCK_ROWS = 2048


def _copy_body(table_ref, out_ref):
    out_ref[...] = table_ref[...]


@jax.jit
def _tc_copy(table):
    return pl.pallas_call(
        _copy_body,
        grid=(MAX_LEN // BLOCK_ROWS,),
        in_specs=[pl.BlockSpec((BLOCK_ROWS, EMBED_DIM), lambda i: (i, 0))],
        out_specs=pl.BlockSpec((BLOCK_ROWS, EMBED_DIM), lambda i: (i, 0)),
        out_shape=jax.ShapeDtypeStruct((MAX_LEN, EMBED_DIM), jnp.float32),
    )(table)


def kernel(x, table):
    del x
    return _tc_copy(table)[None]
